# baseline (speedup 1.0000x reference)
import functools

import jax
import jax.numpy as jnp
from jax import lax
from jax.experimental import pallas as pl
from jax.experimental.pallas import tpu as pltpu
from jax.experimental.pallas import tpu_sc as plsc

STEPS = 5
G = 64
NC = 2
NS = 16
NW = NC * NS
CH = 32
PIECE = 32
NBUF = 8
DEPTH = 4
ROWS = 2000


def _leaky(v):
    return jnp.where(v >= 0, v, 0.01 * v)


def _embed_body(x_ref, wemb_ref, bemb_ref, wcat_ref, eh_ref, tr_ref):
    e = jnp.dot(x_ref[...], wemb_ref[...],
                preferred_element_type=jnp.float32) + bemb_ref[...]
    eh_ref[...] = e
    tr_ref[...] = jnp.dot(e, wcat_ref[...], preferred_element_type=jnp.float32)


def _gru_math(part, h, wih, whh, bih, bhh):
    D = h.shape[1]
    a = part[0] + part[1]
    gi = jnp.dot(a, wih, preferred_element_type=jnp.float32) + bih
    gh = jnp.dot(h, whh, preferred_element_type=jnp.float32) + bhh
    r = jax.nn.sigmoid(gi[:, :D] + gh[:, :D])
    z = jax.nn.sigmoid(gi[:, D:2 * D] + gh[:, D:2 * D])
    n = jnp.tanh(gi[:, 2 * D:] + r * gh[:, 2 * D:])
    return (1.0 - z) * n + z * h


def _gru_body(part_ref, h_ref, wih_ref, whh_ref, bih_ref, bhh_ref, wcat_ref,
              h_out_ref, tr_ref):
    hn = _gru_math(part_ref[...], h_ref[...], wih_ref[...], whh_ref[...],
                   bih_ref[...], bhh_ref[...])
    h_out_ref[...] = hn
    tr_ref[...] = jnp.dot(hn, wcat_ref[...], preferred_element_type=jnp.float32)


def _last_body(part_ref, h_ref, eh_ref, gid_ref, wih_ref, whh_ref, bih_ref,
               bhh_ref, colsum_ref, colsq_ref, seg_ref):
    i = pl.program_id(0)
    hn = _gru_math(part_ref[...], h_ref[...], wih_ref[...], whh_ref[...],
                   bih_ref[...], bhh_ref[...])
    hcat = jnp.concatenate([_leaky(hn), eh_ref[...]], axis=1)
    colsum = jnp.sum(hcat, axis=0, keepdims=True)
    colsq = jnp.sum(hcat * hcat, axis=0, keepdims=True)
    onehot = (gid_ref[...] == lax.broadcasted_iota(jnp.int32, (1, G), 1)
              ).astype(jnp.float32)
    seg = lax.dot_general(onehot, hcat, (((0,), (0,)), ((), ())),
                          preferred_element_type=jnp.float32)

    @pl.when(i == 0)
    def _():
        colsum_ref[...] = colsum
        colsq_ref[...] = colsq
        seg_ref[...] = seg

    @pl.when(i > 0)
    def _():
        colsum_ref[...] += colsum
        colsq_ref[...] += colsq
        seg_ref[...] += seg


def _head_body(seg_ref, colsum_ref, colsq_ref, gid_ref, gamma_ref, beta_ref,
               wf_ref, bf_ref, out_ref, *, n_nodes):
    mu = colsum_ref[...] / n_nodes
    var = colsq_ref[...] / n_nodes - mu * mu
    scale = gamma_ref[...] * jax.lax.rsqrt(var + 1e-5)
    onehot = (gid_ref[...] == lax.broadcasted_iota(jnp.int32, (1, G), 1)
              ).astype(jnp.float32)
    ones = jnp.ones((onehot.shape[0], 1), jnp.float32)
    cnt = lax.dot_general(onehot, ones, (((0,), (0,)), ((), ())),
                          preferred_element_type=jnp.float32)
    mean = seg_ref[...] / jnp.maximum(cnt, 1.0)
    xn = (mean - mu) * scale + beta_ref[...]
    logits = jnp.dot(xn, wf_ref[...], preferred_element_type=jnp.float32)
    logits = _leaky(logits + bf_ref[...])
    m = jnp.max(logits, axis=-1, keepdims=True)
    e = jnp.exp(logits - m)
    out_ref[...] = e / jnp.sum(e, axis=-1, keepdims=True)


def _make_sc_aggregate(n_nodes, n_edges, d):
    sb_total = n_edges // (8 * CH)
    sb_lo = sb_total // NW
    n_hi = sb_total - sb_lo * NW
    max_rows = 8 * (sb_lo + 1)
    zr = ((n_nodes + NS - 1) // NS + 7) // 8 * 8
    zlast = n_nodes - (NS - 1) * zr
    assert zlast > 0 and zlast % 8 == 0
    mesh = plsc.VectorSubcoreMesh(core_axis_name="c", subcore_axis_name="s",
                                  num_cores=NC, num_subcores=NS)

    @functools.partial(
        pl.kernel,
        out_type=jax.ShapeDtypeStruct((NC, n_nodes, d), jnp.float32),
        mesh=mesh,
        scratch_types=[
            pltpu.VMEM((PIECE, CH), jnp.int32),
            pltpu.VMEM((PIECE, CH), jnp.int32),
            [pltpu.VMEM((CH, d), jnp.float32) for _ in range(NBUF)],
            pltpu.VMEM_SHARED((n_nodes, d), jnp.float32),
            [pltpu.SemaphoreType.DMA for _ in range(NBUF)],
            [pltpu.SemaphoreType.DMA for _ in range(NBUF)],
        ],
    )
    def sc_aggregate(trans_hbm, gidx_hbm, dst_hbm, zeros_hbm, out_hbm,
                     gidx_v, dst_v, rows, acc, gsem, ssem):
        c = lax.axis_index("c")
        s = lax.axis_index("s")
        wid = s * NC + c

        @pl.when(s < NS - 1)
        def _():
            pltpu.sync_copy(zeros_hbm.at[pl.ds(s * zr, zr)],
                            acc.at[pl.ds(s * zr, zr)])

        @pl.when(s == NS - 1)
        def _():
            pltpu.sync_copy(zeros_hbm.at[pl.ds((NS - 1) * zr, zlast)],
                            acc.at[pl.ds((NS - 1) * zr, zlast)])

        sbbase = sb_lo * wid + jnp.minimum(wid, n_hi)
        rowbase = 8 * sbbase
        nrows = 8 * (sb_lo + jnp.where(wid < n_hi, 1, 0))

        def wait_gather(j_sl, b):
            pltpu.make_async_copy(trans_hbm.at[gidx_v.at[j_sl]], rows[b],
                                  gsem[b]).wait()

        def drain_scatter(b):
            pltpu.make_async_copy(rows[b], acc.at[dst_v.at[0]],
                                  ssem[b]).wait()

        for piece in range(max_rows // PIECE):
            hb = rowbase + piece * PIECE
            pltpu.sync_copy(gidx_hbm.at[pl.ds(hb, PIECE)], gidx_v)
            pltpu.sync_copy(dst_hbm.at[pl.ds(hb, PIECE)], dst_v)
            hrows = jnp.clip(nrows - piece * PIECE, 0, PIECE)

            @pl.when(hrows > 0)
            def _():
                for t in range(DEPTH):
                    pltpu.async_copy(trans_hbm.at[gidx_v.at[t]], rows[t],
                                     gsem[t])

                def group(g, carry):
                    j0 = NBUF * g
                    for t in range(NBUF):
                        j = j0 + t
                        b = t
                        bg = (t + DEPTH) % NBUF

                        @pl.when(j + DEPTH < hrows)
                        def _():
                            @pl.when(j >= NBUF - DEPTH)
                            def _():
                                drain_scatter(bg)

                            pltpu.async_copy(
                                trans_hbm.at[gidx_v.at[j + DEPTH]],
                                rows[bg], gsem[bg])

                        wait_gather(j, b)
                        pltpu.async_copy(rows[b], acc.at[dst_v.at[j]],
                                         ssem[b], add=True)
                    return carry

                lax.fori_loop(0, hrows // NBUF, group, 0)
                for b in range(NBUF):
                    drain_scatter(b)

        plsc.subcore_barrier()

        @pl.when(s < NS - 1)
        def _():
            pltpu.sync_copy(acc.at[pl.ds(s * zr, zr)],
                            out_hbm.at[c, pl.ds(s * zr, zr)])

        @pl.when(s == NS - 1)
        def _():
            pltpu.sync_copy(acc.at[pl.ds((NS - 1) * zr, zlast)],
                            out_hbm.at[c, pl.ds((NS - 1) * zr, zlast)])

    return sc_aggregate


def _pad_idx(a, n_edges):
    sb_total = n_edges // (8 * CH)
    max_rows = 8 * (sb_total // NW + 1)
    pad_rows = 8 * sb_total + max_rows
    a = jnp.pad(a, (0, pad_rows * CH - n_edges))
    return a.reshape(pad_rows, CH)


def _aggregate(trans2d, g_idx2d, dst2d, zeros_nd, n_edges):
    n_nodes, d = zeros_nd.shape
    fn = _make_sc_aggregate(n_nodes, n_edges, d)
    return fn(trans2d, g_idx2d, dst2d, zeros_nd)


def kernel(node_feat, edge_index, edge_type, graph_ids, W_emb, b_emb, W_edge,
           W_ih, W_hh, b_ih, b_hh, gamma, beta, W_f, b_f):
    N, D = node_feat.shape
    T = W_edge.shape[0]
    C = W_f.shape[1]
    grid = (N // ROWS,)

    E = edge_index.shape[1]
    src, dst = edge_index[0], edge_index[1]
    g_idx2d = _pad_idx(src * T + edge_type, E)
    dst2d = _pad_idx(dst, E)
    W_cat = jnp.transpose(W_edge, (1, 0, 2)).reshape(D, T * D)
    b_ih2, b_hh2 = b_ih.reshape(1, 3 * D), b_hh.reshape(1, 3 * D)
    gid2 = graph_ids.reshape(N, 1)
    zeros_nd = jnp.zeros((N, D), jnp.float32)

    row_spec = lambda w: pl.BlockSpec((ROWS, w), lambda i: (i, 0))
    full = lambda a: pl.BlockSpec(a.shape, lambda i: (0,) * a.ndim)

    embed_h, trans = pl.pallas_call(
        _embed_body,
        grid=grid,
        in_specs=[row_spec(D), full(W_emb), full(b_emb.reshape(1, D)),
                  full(W_cat)],
        out_specs=[row_spec(D), row_spec(T * D)],
        out_shape=[jax.ShapeDtypeStruct((N, D), jnp.float32),
                   jax.ShapeDtypeStruct((N, T * D), jnp.float32)],
    )(node_feat, W_emb, b_emb.reshape(1, D), W_cat)

    part_spec = pl.BlockSpec((NC, ROWS, D), lambda i: (0, i, 0))
    h = embed_h
    for _ in range(STEPS - 1):
        part = _aggregate(trans.reshape(N * T, D), g_idx2d, dst2d, zeros_nd, E)
        h, trans = pl.pallas_call(
            _gru_body,
            grid=grid,
            in_specs=[part_spec, row_spec(D), full(W_ih), full(W_hh),
                      full(b_ih2), full(b_hh2), full(W_cat)],
            out_specs=[row_spec(D), row_spec(T * D)],
            out_shape=[jax.ShapeDtypeStruct((N, D), jnp.float32),
                       jax.ShapeDtypeStruct((N, T * D), jnp.float32)],
        )(part, h, W_ih, W_hh, b_ih2, b_hh2, W_cat)

    part = _aggregate(trans.reshape(N * T, D), g_idx2d, dst2d, zeros_nd, E)
    colsum, colsq, seg = pl.pallas_call(
        _last_body,
        grid=grid,
        in_specs=[part_spec, row_spec(D), row_spec(D),
                  pl.BlockSpec((ROWS, 1), lambda i: (i, 0)),
                  full(W_ih), full(W_hh), full(b_ih2), full(b_hh2)],
        out_specs=[pl.BlockSpec((1, 2 * D), lambda i: (0, 0)),
                   pl.BlockSpec((1, 2 * D), lambda i: (0, 0)),
                   pl.BlockSpec((G, 2 * D), lambda i: (0, 0))],
        out_shape=[jax.ShapeDtypeStruct((1, 2 * D), jnp.float32),
                   jax.ShapeDtypeStruct((1, 2 * D), jnp.float32),
                   jax.ShapeDtypeStruct((G, 2 * D), jnp.float32)],
    )(part, h, embed_h, gid2, W_ih, W_hh, b_ih2, b_hh2)

    out = pl.pallas_call(
        functools.partial(_head_body, n_nodes=float(N)),
        in_specs=[pl.BlockSpec(x.shape, lambda n=x.ndim: (0,) * n) for x in
                  (seg, colsum, colsq, gid2,
                   gamma.reshape(1, 2 * D), beta.reshape(1, 2 * D),
                   W_f, b_f.reshape(1, C))],
        out_specs=pl.BlockSpec((G, C), lambda: (0, 0)),
        out_shape=jax.ShapeDtypeStruct((G, C), jnp.float32),
    )(seg, colsum, colsq, gid2, gamma.reshape(1, 2 * D),
      beta.reshape(1, 2 * D), W_f, b_f.reshape(1, C))
    return out


# speedup vs baseline: 1.0889x; 1.0889x over previous
import functools

import jax
import jax.numpy as jnp
from jax import lax
from jax.experimental import pallas as pl
from jax.experimental.pallas import tpu as pltpu
from jax.experimental.pallas import tpu_sc as plsc

STEPS = 5
G = 64
NC = 2
NS = 16
NW = NC * NS
CH = 64
PIECE = 32
NBUF = 4
DEPTH = 2
ROWS = 2000


def _leaky(v):
    return jnp.where(v >= 0, v, 0.01 * v)


def _embed_body(x_ref, wemb_ref, bemb_ref, wcat_ref, eh_ref, tr_ref):
    e = jnp.dot(x_ref[...], wemb_ref[...],
                preferred_element_type=jnp.float32) + bemb_ref[...]
    eh_ref[...] = e
    tr_ref[...] = jnp.dot(e, wcat_ref[...], preferred_element_type=jnp.float32)


def _gru_math(part, h, wih, whh, bih, bhh):
    D = h.shape[1]
    a = part[0] + part[1]
    gi = jnp.dot(a, wih, preferred_element_type=jnp.float32) + bih
    gh = jnp.dot(h, whh, preferred_element_type=jnp.float32) + bhh
    r = jax.nn.sigmoid(gi[:, :D] + gh[:, :D])
    z = jax.nn.sigmoid(gi[:, D:2 * D] + gh[:, D:2 * D])
    n = jnp.tanh(gi[:, 2 * D:] + r * gh[:, 2 * D:])
    return (1.0 - z) * n + z * h


def _gru_body(part_ref, h_ref, wih_ref, whh_ref, bih_ref, bhh_ref, wcat_ref,
              h_out_ref, tr_ref):
    hn = _gru_math(part_ref[...], h_ref[...], wih_ref[...], whh_ref[...],
                   bih_ref[...], bhh_ref[...])
    h_out_ref[...] = hn
    tr_ref[...] = jnp.dot(hn, wcat_ref[...], preferred_element_type=jnp.float32)


def _last_body(part_ref, h_ref, eh_ref, gid_ref, wih_ref, whh_ref, bih_ref,
               bhh_ref, gamma_ref, beta_ref, wf_ref, bf_ref,
               colsum_ref, colsq_ref, seg_ref, cnt_ref, out_ref, *, n_nodes):
    i = pl.program_id(0)
    n_blocks = pl.num_programs(0)
    hn = _gru_math(part_ref[...], h_ref[...], wih_ref[...], whh_ref[...],
                   bih_ref[...], bhh_ref[...])
    hcat = jnp.concatenate([_leaky(hn), eh_ref[...]], axis=1)
    colsum = jnp.sum(hcat, axis=0, keepdims=True)
    colsq = jnp.sum(hcat * hcat, axis=0, keepdims=True)
    onehot = (gid_ref[...] == lax.broadcasted_iota(jnp.int32, (1, G), 1)
              ).astype(jnp.float32)
    seg = lax.dot_general(onehot, hcat, (((0,), (0,)), ((), ())),
                          preferred_element_type=jnp.float32)
    ones = jnp.ones((onehot.shape[0], 1), jnp.float32)
    cnt = lax.dot_general(onehot, ones, (((0,), (0,)), ((), ())),
                          preferred_element_type=jnp.float32)

    @pl.when(i == 0)
    def _():
        colsum_ref[...] = colsum
        colsq_ref[...] = colsq
        seg_ref[...] = seg
        cnt_ref[...] = cnt

    @pl.when(i > 0)
    def _():
        colsum_ref[...] += colsum
        colsq_ref[...] += colsq
        seg_ref[...] += seg
        cnt_ref[...] += cnt

    @pl.when(i == n_blocks - 1)
    def _():
        mu = colsum_ref[...] / n_nodes
        var = colsq_ref[...] / n_nodes - mu * mu
        scale = gamma_ref[...] * jax.lax.rsqrt(var + 1e-5)
        mean = seg_ref[...] / jnp.maximum(cnt_ref[...], 1.0)
        xn = (mean - mu) * scale + beta_ref[...]
        logits = jnp.dot(xn, wf_ref[...], preferred_element_type=jnp.float32)
        logits = _leaky(logits + bf_ref[...])
        m = jnp.max(logits, axis=-1, keepdims=True)
        e = jnp.exp(logits - m)
        out_ref[...] = e / jnp.sum(e, axis=-1, keepdims=True)


@functools.lru_cache(maxsize=None)
def _make_sc_aggregate(n_nodes, n_edges, d):
    sb_total = n_edges // (8 * CH)
    sb_lo = sb_total // NW
    n_hi = sb_total - sb_lo * NW
    max_rows = 8 * (sb_lo + 1)
    zr = ((n_nodes + NS - 1) // NS + 7) // 8 * 8
    zlast = n_nodes - (NS - 1) * zr
    assert zlast > 0 and zlast % 8 == 0
    mesh = plsc.VectorSubcoreMesh(core_axis_name="c", subcore_axis_name="s",
                                  num_cores=NC, num_subcores=NS)

    @functools.partial(
        pl.kernel,
        out_type=jax.ShapeDtypeStruct((NC, n_nodes, d), jnp.float32),
        mesh=mesh,
        scratch_types=[
            pltpu.VMEM((PIECE, CH), jnp.int32),
            pltpu.VMEM((PIECE, CH), jnp.int32),
            [pltpu.VMEM((CH, d), jnp.float32) for _ in range(NBUF)],
            pltpu.VMEM_SHARED((n_nodes, d), jnp.float32),
            [pltpu.SemaphoreType.DMA for _ in range(NBUF)],
            [pltpu.SemaphoreType.DMA for _ in range(NBUF)],
        ],
    )
    def sc_aggregate(trans_hbm, gidx_hbm, dst_hbm, zeros_hbm, out_hbm,
                     gidx_v, dst_v, rows, acc, gsem, ssem):
        c = lax.axis_index("c")
        s = lax.axis_index("s")
        wid = s * NC + c

        @pl.when(s < NS - 1)
        def _():
            pltpu.sync_copy(zeros_hbm.at[pl.ds(s * zr, zr)],
                            acc.at[pl.ds(s * zr, zr)])

        @pl.when(s == NS - 1)
        def _():
            pltpu.sync_copy(zeros_hbm.at[pl.ds((NS - 1) * zr, zlast)],
                            acc.at[pl.ds((NS - 1) * zr, zlast)])

        sbbase = sb_lo * wid + jnp.minimum(wid, n_hi)
        rowbase = 8 * sbbase
        nrows = 8 * (sb_lo + jnp.where(wid < n_hi, 1, 0))

        def wait_gather(j_sl, b):
            pltpu.make_async_copy(trans_hbm.at[gidx_v.at[j_sl]], rows[b],
                                  gsem[b]).wait()

        def drain_scatter(b):
            pltpu.make_async_copy(rows[b], acc.at[dst_v.at[0]],
                                  ssem[b]).wait()

        def piece_body(piece, carry):
            hb = rowbase + piece * PIECE
            pltpu.sync_copy(gidx_hbm.at[pl.ds(hb, PIECE)], gidx_v)
            pltpu.sync_copy(dst_hbm.at[pl.ds(hb, PIECE)], dst_v)
            hrows = jnp.clip(nrows - piece * PIECE, 0, PIECE)

            @pl.when(hrows > 0)
            def _():
                for t in range(DEPTH):
                    pltpu.async_copy(trans_hbm.at[gidx_v.at[t]], rows[t],
                                     gsem[t])

                def group(g, carry):
                    j0 = NBUF * g
                    for t in range(NBUF):
                        j = j0 + t
                        b = t
                        bg = (t + DEPTH) % NBUF

                        @pl.when(j + DEPTH < hrows)
                        def _():
                            @pl.when(j >= NBUF - DEPTH)
                            def _():
                                drain_scatter(bg)

                            pltpu.async_copy(
                                trans_hbm.at[gidx_v.at[j + DEPTH]],
                                rows[bg], gsem[bg])

                        wait_gather(j, b)
                        pltpu.async_copy(rows[b], acc.at[dst_v.at[j]],
                                         ssem[b], add=True)
                    return carry

                lax.fori_loop(0, hrows // NBUF, group, 0)
                for b in range(NBUF):
                    drain_scatter(b)

            return carry

        lax.fori_loop(0, max_rows // PIECE, piece_body, 0)
        plsc.subcore_barrier()

        @pl.when(s < NS - 1)
        def _():
            pltpu.sync_copy(acc.at[pl.ds(s * zr, zr)],
                            out_hbm.at[c, pl.ds(s * zr, zr)])

        @pl.when(s == NS - 1)
        def _():
            pltpu.sync_copy(acc.at[pl.ds((NS - 1) * zr, zlast)],
                            out_hbm.at[c, pl.ds((NS - 1) * zr, zlast)])

    return sc_aggregate


def _pad_idx(a, n_edges):
    sb_total = n_edges // (8 * CH)
    max_rows = 8 * (sb_total // NW + 1)
    pad_rows = 8 * sb_total + max_rows
    a = jnp.pad(a, (0, pad_rows * CH - n_edges))
    return a.reshape(pad_rows, CH)


def _aggregate(trans2d, g_idx2d, dst2d, zeros_nd, n_edges):
    n_nodes, d = zeros_nd.shape
    fn = _make_sc_aggregate(n_nodes, n_edges, d)
    return fn(trans2d, g_idx2d, dst2d, zeros_nd)


def kernel(node_feat, edge_index, edge_type, graph_ids, W_emb, b_emb, W_edge,
           W_ih, W_hh, b_ih, b_hh, gamma, beta, W_f, b_f):
    N, D = node_feat.shape
    T = W_edge.shape[0]
    C = W_f.shape[1]
    grid = (N // ROWS,)

    E = edge_index.shape[1]
    src, dst = edge_index[0], edge_index[1]
    g_idx2d = _pad_idx(src * T + edge_type, E)
    dst2d = _pad_idx(dst, E)
    W_cat = jnp.transpose(W_edge, (1, 0, 2)).reshape(D, T * D)
    b_ih2, b_hh2 = b_ih.reshape(1, 3 * D), b_hh.reshape(1, 3 * D)
    gid2 = graph_ids.reshape(N, 1)
    zeros_nd = jnp.zeros((N, D), jnp.float32)

    row_spec = lambda w: pl.BlockSpec((ROWS, w), lambda i: (i, 0))
    full = lambda a: pl.BlockSpec(a.shape, lambda i: (0,) * a.ndim)

    embed_h, trans = pl.pallas_call(
        _embed_body,
        grid=grid,
        in_specs=[row_spec(D), full(W_emb), full(b_emb.reshape(1, D)),
                  full(W_cat)],
        out_specs=[row_spec(D), row_spec(T * D)],
        out_shape=[jax.ShapeDtypeStruct((N, D), jnp.float32),
                   jax.ShapeDtypeStruct((N, T * D), jnp.float32)],
    )(node_feat, W_emb, b_emb.reshape(1, D), W_cat)

    part_spec = pl.BlockSpec((NC, ROWS, D), lambda i: (0, i, 0))
    h = embed_h
    for _ in range(STEPS - 1):
        part = _aggregate(trans.reshape(N * T, D), g_idx2d, dst2d, zeros_nd, E)
        h, trans = pl.pallas_call(
            _gru_body,
            grid=grid,
            in_specs=[part_spec, row_spec(D), full(W_ih), full(W_hh),
                      full(b_ih2), full(b_hh2), full(W_cat)],
            out_specs=[row_spec(D), row_spec(T * D)],
            out_shape=[jax.ShapeDtypeStruct((N, D), jnp.float32),
                       jax.ShapeDtypeStruct((N, T * D), jnp.float32)],
        )(part, h, W_ih, W_hh, b_ih2, b_hh2, W_cat)

    part = _aggregate(trans.reshape(N * T, D), g_idx2d, dst2d, zeros_nd, E)
    const = lambda shp: pl.BlockSpec(shp, lambda i: (0,) * len(shp))
    outs = pl.pallas_call(
        functools.partial(_last_body, n_nodes=float(N)),
        grid=grid,
        in_specs=[part_spec, row_spec(D), row_spec(D),
                  pl.BlockSpec((ROWS, 1), lambda i: (i, 0)),
                  full(W_ih), full(W_hh), full(b_ih2), full(b_hh2),
                  full(gamma.reshape(1, 2 * D)), full(beta.reshape(1, 2 * D)),
                  full(W_f), full(b_f.reshape(1, C))],
        out_specs=[const((1, 2 * D)), const((1, 2 * D)), const((G, 2 * D)),
                   const((G, 1)), const((G, C))],
        out_shape=[jax.ShapeDtypeStruct((1, 2 * D), jnp.float32),
                   jax.ShapeDtypeStruct((1, 2 * D), jnp.float32),
                   jax.ShapeDtypeStruct((G, 2 * D), jnp.float32),
                   jax.ShapeDtypeStruct((G, 1), jnp.float32),
                   jax.ShapeDtypeStruct((G, C), jnp.float32)],
    )(part, h, embed_h, gid2, W_ih, W_hh, b_ih2, b_hh2,
      gamma.reshape(1, 2 * D), beta.reshape(1, 2 * D), W_f,
      b_f.reshape(1, C))
    return outs[4]


# speedup vs baseline: 1.2404x; 1.1391x over previous
import functools

import jax
import jax.numpy as jnp
from jax import lax
from jax.experimental import pallas as pl
from jax.experimental.pallas import tpu as pltpu
from jax.experimental.pallas import tpu_sc as plsc

STEPS = 5
G = 64
NC = 2
NS = 16
NW = NC * NS
CH = 64
PIECE = 32
NBUF = 4
DEPTH = 2
ROWS = 2000


def _leaky(v):
    return jnp.where(v >= 0, v, 0.01 * v)


def _embed_body(x_ref, wemb_ref, bemb_ref, wcat_ref, eh_ref, tr_ref):
    e = jnp.dot(x_ref[...], wemb_ref[...],
                preferred_element_type=jnp.float32) + bemb_ref[...]
    eh_ref[...] = e
    tr = jnp.dot(e, wcat_ref[...], preferred_element_type=jnp.float32)
    tr_ref[...] = tr.reshape(tr_ref.shape)


def _gru_math(part, h, wih, whh, bih, bhh):
    D = h.shape[1]
    a = part[0] + part[1]
    gi = jnp.dot(a, wih, preferred_element_type=jnp.float32) + bih
    gh = jnp.dot(h, whh, preferred_element_type=jnp.float32) + bhh
    r = jax.nn.sigmoid(gi[:, :D] + gh[:, :D])
    z = jax.nn.sigmoid(gi[:, D:2 * D] + gh[:, D:2 * D])
    n = jnp.tanh(gi[:, 2 * D:] + r * gh[:, 2 * D:])
    return (1.0 - z) * n + z * h


def _gru_body(part_ref, h_ref, wih_ref, whh_ref, bih_ref, bhh_ref, wcat_ref,
              h_out_ref, tr_ref):
    hn = _gru_math(part_ref[...], h_ref[...], wih_ref[...], whh_ref[...],
                   bih_ref[...], bhh_ref[...])
    h_out_ref[...] = hn
    tr = jnp.dot(hn, wcat_ref[...], preferred_element_type=jnp.float32)
    tr_ref[...] = tr.reshape(tr_ref.shape)


def _last_body(part_ref, h_ref, eh_ref, gid_ref, wih_ref, whh_ref, bih_ref,
               bhh_ref, gamma_ref, beta_ref, wf_ref, bf_ref,
               colsum_ref, colsq_ref, seg_ref, cnt_ref, out_ref, *, n_nodes):
    i = pl.program_id(0)
    n_blocks = pl.num_programs(0)
    hn = _gru_math(part_ref[...], h_ref[...], wih_ref[...], whh_ref[...],
                   bih_ref[...], bhh_ref[...])
    hcat = jnp.concatenate([_leaky(hn), eh_ref[...]], axis=1)
    colsum = jnp.sum(hcat, axis=0, keepdims=True)
    colsq = jnp.sum(hcat * hcat, axis=0, keepdims=True)
    onehot = (gid_ref[...] == lax.broadcasted_iota(jnp.int32, (1, G), 1)
              ).astype(jnp.float32)
    seg = lax.dot_general(onehot, hcat, (((0,), (0,)), ((), ())),
                          preferred_element_type=jnp.float32)
    ones = jnp.ones((onehot.shape[0], 1), jnp.float32)
    cnt = lax.dot_general(onehot, ones, (((0,), (0,)), ((), ())),
                          preferred_element_type=jnp.float32)

    @pl.when(i == 0)
    def _():
        colsum_ref[...] = colsum
        colsq_ref[...] = colsq
        seg_ref[...] = seg
        cnt_ref[...] = cnt

    @pl.when(i > 0)
    def _():
        colsum_ref[...] += colsum
        colsq_ref[...] += colsq
        seg_ref[...] += seg
        cnt_ref[...] += cnt

    @pl.when(i == n_blocks - 1)
    def _():
        mu = colsum_ref[...] / n_nodes
        var = colsq_ref[...] / n_nodes - mu * mu
        scale = gamma_ref[...] * jax.lax.rsqrt(var + 1e-5)
        mean = seg_ref[...] / jnp.maximum(cnt_ref[...], 1.0)
        xn = (mean - mu) * scale + beta_ref[...]
        logits = jnp.dot(xn, wf_ref[...], preferred_element_type=jnp.float32)
        logits = _leaky(logits + bf_ref[...])
        m = jnp.max(logits, axis=-1, keepdims=True)
        e = jnp.exp(logits - m)
        out_ref[...] = e / jnp.sum(e, axis=-1, keepdims=True)


@functools.lru_cache(maxsize=None)
def _make_sc_aggregate(n_nodes, n_edges, d):
    sb_total = n_edges // (8 * CH)
    sb_lo = sb_total // NW
    n_hi = sb_total - sb_lo * NW
    max_rows = 8 * (sb_lo + 1)
    zr = ((n_nodes + NS - 1) // NS + 7) // 8 * 8
    zlast = n_nodes - (NS - 1) * zr
    assert zlast > 0 and zlast % 8 == 0
    mesh = plsc.VectorSubcoreMesh(core_axis_name="c", subcore_axis_name="s",
                                  num_cores=NC, num_subcores=NS)

    @functools.partial(
        pl.kernel,
        out_type=jax.ShapeDtypeStruct((NC, n_nodes, d), jnp.float32),
        mesh=mesh,
        scratch_types=[
            pltpu.VMEM((PIECE, CH), jnp.int32),
            pltpu.VMEM((PIECE, CH), jnp.int32),
            [pltpu.VMEM((CH, d), jnp.float32) for _ in range(NBUF)],
            pltpu.VMEM_SHARED((n_nodes, d), jnp.float32),
            [pltpu.SemaphoreType.DMA for _ in range(NBUF)],
            [pltpu.SemaphoreType.DMA for _ in range(NBUF)],
        ],
    )
    def sc_aggregate(trans_hbm, gidx_hbm, dst_hbm, zeros_hbm, out_hbm,
                     gidx_v, dst_v, rows, acc, gsem, ssem):
        c = lax.axis_index("c")
        s = lax.axis_index("s")
        wid = s * NC + c

        @pl.when(s < NS - 1)
        def _():
            pltpu.sync_copy(zeros_hbm.at[pl.ds(s * zr, zr)],
                            acc.at[pl.ds(s * zr, zr)])

        @pl.when(s == NS - 1)
        def _():
            pltpu.sync_copy(zeros_hbm.at[pl.ds((NS - 1) * zr, zlast)],
                            acc.at[pl.ds((NS - 1) * zr, zlast)])

        sbbase = sb_lo * wid + jnp.minimum(wid, n_hi)
        rowbase = 8 * sbbase
        nrows = 8 * (sb_lo + jnp.where(wid < n_hi, 1, 0))

        def wait_gather(j_sl, b):
            pltpu.make_async_copy(trans_hbm.at[gidx_v.at[j_sl]], rows[b],
                                  gsem[b]).wait()

        def drain_scatter(b):
            pltpu.make_async_copy(rows[b], acc.at[dst_v.at[0]],
                                  ssem[b]).wait()

        def piece_body(piece, carry):
            hb = rowbase + piece * PIECE
            pltpu.sync_copy(gidx_hbm.at[pl.ds(hb, PIECE)], gidx_v)
            pltpu.sync_copy(dst_hbm.at[pl.ds(hb, PIECE)], dst_v)
            hrows = jnp.clip(nrows - piece * PIECE, 0, PIECE)

            @pl.when(hrows > 0)
            def _():
                for t in range(DEPTH):
                    pltpu.async_copy(trans_hbm.at[gidx_v.at[t]], rows[t],
                                     gsem[t])

                def group(g, carry):
                    j0 = NBUF * g
                    for t in range(NBUF):
                        j = j0 + t
                        b = t
                        bg = (t + DEPTH) % NBUF

                        @pl.when(j + DEPTH < hrows)
                        def _():
                            @pl.when(j >= NBUF - DEPTH)
                            def _():
                                drain_scatter(bg)

                            pltpu.async_copy(
                                trans_hbm.at[gidx_v.at[j + DEPTH]],
                                rows[bg], gsem[bg])

                        wait_gather(j, b)
                        pltpu.async_copy(rows[b], acc.at[dst_v.at[j]],
                                         ssem[b], add=True)
                    return carry

                lax.fori_loop(0, hrows // NBUF, group, 0)
                for b in range(NBUF):
                    drain_scatter(b)

            return carry

        lax.fori_loop(0, max_rows // PIECE, piece_body, 0)
        plsc.subcore_barrier()

        @pl.when(s < NS - 1)
        def _():
            pltpu.sync_copy(acc.at[pl.ds(s * zr, zr)],
                            out_hbm.at[c, pl.ds(s * zr, zr)])

        @pl.when(s == NS - 1)
        def _():
            pltpu.sync_copy(acc.at[pl.ds((NS - 1) * zr, zlast)],
                            out_hbm.at[c, pl.ds((NS - 1) * zr, zlast)])

    return sc_aggregate


def _pad_idx(a, n_edges):
    sb_total = n_edges // (8 * CH)
    max_rows = 8 * (sb_total // NW + 1)
    pad_rows = 8 * sb_total + max_rows
    a = jnp.pad(a, (0, pad_rows * CH - n_edges))
    return a.reshape(pad_rows, CH)


def _aggregate(trans2d, g_idx2d, dst2d, zeros_nd, n_edges):
    n_nodes, d = zeros_nd.shape
    fn = _make_sc_aggregate(n_nodes, n_edges, d)
    return fn(trans2d, g_idx2d, dst2d, zeros_nd)


def kernel(node_feat, edge_index, edge_type, graph_ids, W_emb, b_emb, W_edge,
           W_ih, W_hh, b_ih, b_hh, gamma, beta, W_f, b_f):
    N, D = node_feat.shape
    T = W_edge.shape[0]
    C = W_f.shape[1]
    grid = (N // ROWS,)

    E = edge_index.shape[1]
    src, dst = edge_index[0], edge_index[1]
    g_idx2d = _pad_idx(src * T + edge_type, E)
    dst2d = _pad_idx(dst, E)
    W_cat = jnp.transpose(W_edge, (1, 0, 2)).reshape(D, T * D)
    b_ih2, b_hh2 = b_ih.reshape(1, 3 * D), b_hh.reshape(1, 3 * D)
    gid2 = graph_ids.reshape(N, 1)
    zeros_nd = jnp.zeros((N, D), jnp.float32)

    row_spec = lambda w: pl.BlockSpec((ROWS, w), lambda i: (i, 0))
    tr_spec = pl.BlockSpec((ROWS * T, D), lambda i: (i, 0))
    tr_shape = jax.ShapeDtypeStruct((N * T, D), jnp.float32)
    full = lambda a: pl.BlockSpec(a.shape, lambda i: (0,) * a.ndim)

    embed_h, trans = pl.pallas_call(
        _embed_body,
        grid=grid,
        in_specs=[row_spec(D), full(W_emb), full(b_emb.reshape(1, D)),
                  full(W_cat)],
        out_specs=[row_spec(D), tr_spec],
        out_shape=[jax.ShapeDtypeStruct((N, D), jnp.float32), tr_shape],
    )(node_feat, W_emb, b_emb.reshape(1, D), W_cat)

    part_spec = pl.BlockSpec((NC, ROWS, D), lambda i: (0, i, 0))
    h = embed_h
    for _ in range(STEPS - 1):
        part = _aggregate(trans, g_idx2d, dst2d, zeros_nd, E)
        h, trans = pl.pallas_call(
            _gru_body,
            grid=grid,
            in_specs=[part_spec, row_spec(D), full(W_ih), full(W_hh),
                      full(b_ih2), full(b_hh2), full(W_cat)],
            out_specs=[row_spec(D), tr_spec],
            out_shape=[jax.ShapeDtypeStruct((N, D), jnp.float32), tr_shape],
        )(part, h, W_ih, W_hh, b_ih2, b_hh2, W_cat)

    part = _aggregate(trans, g_idx2d, dst2d, zeros_nd, E)
    const = lambda shp: pl.BlockSpec(shp, lambda i: (0,) * len(shp))
    outs = pl.pallas_call(
        functools.partial(_last_body, n_nodes=float(N)),
        grid=grid,
        in_specs=[part_spec, row_spec(D), row_spec(D),
                  pl.BlockSpec((ROWS, 1), lambda i: (i, 0)),
                  full(W_ih), full(W_hh), full(b_ih2), full(b_hh2),
                  full(gamma.reshape(1, 2 * D)), full(beta.reshape(1, 2 * D)),
                  full(W_f), full(b_f.reshape(1, C))],
        out_specs=[const((1, 2 * D)), const((1, 2 * D)), const((G, 2 * D)),
                   const((G, 1)), const((G, C))],
        out_shape=[jax.ShapeDtypeStruct((1, 2 * D), jnp.float32),
                   jax.ShapeDtypeStruct((1, 2 * D), jnp.float32),
                   jax.ShapeDtypeStruct((G, 2 * D), jnp.float32),
                   jax.ShapeDtypeStruct((G, 1), jnp.float32),
                   jax.ShapeDtypeStruct((G, C), jnp.float32)],
    )(part, h, embed_h, gid2, W_ih, W_hh, b_ih2, b_hh2,
      gamma.reshape(1, 2 * D), beta.reshape(1, 2 * D), W_f,
      b_f.reshape(1, C))
    return outs[4]
